# Initial kernel scaffold; baseline (speedup 1.0000x reference)
import jax
import jax.numpy as jnp
from jax.experimental import pallas as pl


def kernel(q, pc, seg_W1, seg_b1, seg_W2, seg_b2, seg_W3, seg_b3, enc_W1, enc_b1, enc_W2, enc_b2, enc_W3, enc_b3, dec_Wq, dec_Wc, dec_b1, dec_W2, dec_b2, dec_W3, dec_b3):
    raise NotImplementedError("write your pallas kernel here")


# speedup vs baseline: 9.5778x; 9.5778x over previous
import jax
import jax.numpy as jnp
from jax import lax
from jax.experimental import pallas as pl
from jax.experimental.pallas import tpu as pltpu

B = 8
N_POINTS = 2048
N_SAMPLE = 2048
DIM = 3
C_DIM = 128
N_CLASSES = 8
H_SEG = 128
H_ENC = 128
H_DEC = 256


def _onet_kernel(q_ref, pc_ref, sW1, sb1, sW2, sb2, sW3, sb3,
                 eW1, eb1, eW2, eb2, eW3, eb3,
                 dWq, dWc, db1, dW2, db2, dW3, db3, out_ref):
    pc = pc_ref[0]
    qb = q_ref[0]

    f32 = jnp.float32

    h = jnp.maximum(jnp.dot(pc, sW1[...], preferred_element_type=f32) + sb1[...], 0.0)
    h = jnp.maximum(jnp.dot(h, sW2[...], preferred_element_type=f32) + sb2[...], 0.0)
    logits = jnp.dot(h, sW3[...], preferred_element_type=f32) + sb3[...]

    m = jnp.max(logits, axis=1, keepdims=True)
    iota = lax.broadcasted_iota(jnp.int32, (N_POINTS, N_CLASSES), 1)
    tag = jnp.min(jnp.where(logits == m, iota, N_CLASSES), axis=1, keepdims=True)
    oh = (tag == iota).astype(f32)

    e = jnp.maximum(jnp.dot(pc, eW1[...], preferred_element_type=f32) + eb1[...], 0.0)
    h2 = jnp.maximum(jnp.dot(e, eW2[...], preferred_element_type=f32) + eb2[...], 0.0)

    counts = jnp.sum(oh, axis=0, keepdims=True)
    ohn = oh / jnp.maximum(counts, 1.0)
    pooled = lax.dot_general(ohn, h2, (((0,), (0,)), ((), ())),
                             preferred_element_type=f32)
    code = jnp.dot(pooled, eW3[...], preferred_element_type=f32) + eb3[...]

    qW = jnp.dot(qb, dWq[...], preferred_element_type=f32)
    cW = jnp.dot(code, dWc[...], preferred_element_type=f32) + db1[...]

    cols = []
    for t in range(N_CLASSES):
        h1 = jnp.maximum(qW + cW[t:t + 1, :], 0.0)
        hh = jnp.maximum(jnp.dot(h1, dW2[...], preferred_element_type=f32) + db2[...], 0.0)
        cols.append(jnp.dot(hh, dW3[...], preferred_element_type=f32) + db3[...])
    out_ref[0] = jnp.concatenate(cols, axis=1)


def kernel(q, pc, seg_W1, seg_b1, seg_W2, seg_b2, seg_W3, seg_b3,
           enc_W1, enc_b1, enc_W2, enc_b2, enc_W3, enc_b3,
           dec_Wq, dec_Wc, dec_b1, dec_W2, dec_b2, dec_W3, dec_b3):
    f32 = jnp.float32
    weights = [seg_W1, seg_b1.reshape(1, -1), seg_W2, seg_b2.reshape(1, -1),
               seg_W3, seg_b3.reshape(1, -1),
               enc_W1, enc_b1.reshape(1, -1), enc_W2, enc_b2.reshape(1, -1),
               enc_W3, enc_b3.reshape(1, -1),
               dec_Wq, dec_Wc, dec_b1.reshape(1, -1), dec_W2, dec_b2.reshape(1, -1),
               dec_W3, dec_b3.reshape(1, -1)]

    def wspec(w):
        return pl.BlockSpec(w.shape, lambda b: (0,) * w.ndim)

    grid_spec = pl.GridSpec(
        grid=(B,),
        in_specs=[pl.BlockSpec((1, N_SAMPLE, DIM), lambda b: (b, 0, 0)),
                  pl.BlockSpec((1, N_POINTS, DIM), lambda b: (b, 0, 0))]
                 + [wspec(w) for w in weights],
        out_specs=pl.BlockSpec((1, N_SAMPLE, N_CLASSES), lambda b: (b, 0, 0)),
    )

    out = pl.pallas_call(
        _onet_kernel,
        grid_spec=grid_spec,
        out_shape=jax.ShapeDtypeStruct((B, N_SAMPLE, N_CLASSES), f32),
        compiler_params=pltpu.CompilerParams(
            dimension_semantics=("parallel",),
        ),
    )(q, pc, *weights)
    return jnp.transpose(out, (0, 2, 1))
